# baseline (speedup 1.0000x reference)
import functools

import jax
import jax.numpy as jnp
from jax import lax
from jax.experimental import pallas as pl
from jax.experimental.pallas import tpu as pltpu
from jax.experimental.pallas import tpu_sc as plsc

_NUM_CORES = 2
_NUM_SUBCORES = 16
_NUM_WORKERS = _NUM_CORES * _NUM_SUBCORES
_NBUF = 3
_LOOKAHEAD = 2


def _gather_body(n_chunks, chunk, width, table_hbm, idx_hbm, out_hbm,
                 idx_v, jdx_v, pairs_v, comp_v, gsem, wsem):
    wid = lax.axis_index("s") * _NUM_CORES + lax.axis_index("c")
    base = wid * (n_chunks * chunk)

    def stage_idx(j, b):
        pltpu.sync_copy(idx_hbm.at[pl.ds(base + j * chunk, chunk)],
                        idx_v.at[b])
        for v in range(chunk // 16):
            sl = pl.ds(v * 16, 16)
            jdx_v[b, sl] = lax.shift_right_logical(idx_v[b, sl], 1)

    def fire_gather(b):
        pltpu.async_copy(table_hbm.at[jdx_v.at[b]], pairs_v.at[b],
                         gsem.at[b])

    def wait_gather(b):
        pltpu.make_async_copy(table_hbm.at[jdx_v.at[b]], pairs_v.at[b],
                              gsem.at[b]).wait()

    def select(b):
        def group(g, carry):
            k0 = g * 16
            hvec = (idx_v[b, pl.ds(k0, 16)] & 1) * width
            for l in range(16):
                off = hvec[l]
                for c in range(width // 16):
                    comp_v[b, k0 + l, pl.ds(c * 16, 16)] = pairs_v[
                        b, k0 + l, pl.ds(off + c * 16, 16)]
            return carry
        lax.fori_loop(0, chunk // 16, group, 0)

    def fire_wb(j, b):
        pltpu.async_copy(comp_v.at[b], out_hbm.at[pl.ds(base + j * chunk,
                                                        chunk)], wsem.at[b])

    def wait_wb(j, b):
        pltpu.make_async_copy(comp_v.at[b],
                              out_hbm.at[pl.ds(base + j * chunk, chunk)],
                              wsem.at[b]).wait()

    for j in range(_LOOKAHEAD):
        stage_idx(j, j)
        fire_gather(j)

    def chunk_step(j, b, bn, *, do_fire, do_wait_wb):
        wait_gather(b)
        if do_fire:
            stage_idx(j + _LOOKAHEAD, bn)
            fire_gather(bn)
        if do_wait_wb:
            wait_wb(j - _NBUF, b)
        select(b)
        fire_wb(j, b)

    for j in range(_NBUF):
        chunk_step(j, j % _NBUF, (j + _LOOKAHEAD) % _NBUF,
                   do_fire=True, do_wait_wb=False)

    def step(jo, carry):
        j0 = jo * _NBUF
        for b in range(_NBUF):
            chunk_step(j0 + b, b, (b + _LOOKAHEAD) % _NBUF,
                       do_fire=True, do_wait_wb=True)
        return carry

    lax.fori_loop(1, (n_chunks - _LOOKAHEAD) // _NBUF, step, 0)

    for j in range(n_chunks - _LOOKAHEAD, n_chunks):
        b = j % _NBUF
        chunk_step(j, b, (b + _LOOKAHEAD) % _NBUF,
                   do_fire=False, do_wait_wb=True)
    for j in range(n_chunks - _NBUF, n_chunks):
        wait_wb(j, j % _NBUF)


@functools.partial(jax.jit, static_argnames=("n_rows", "chunk"))
def _sc_embedding_lookup(idx_flat, table_pairs, *, n_rows, chunk):
    pair_width = table_pairs.shape[1]
    width = pair_width // 2
    per_worker = n_rows // _NUM_WORKERS
    n_chunks = per_worker // chunk
    mesh = plsc.VectorSubcoreMesh(
        core_axis_name="c", subcore_axis_name="s",
        num_cores=_NUM_CORES, num_subcores=_NUM_SUBCORES)
    body = functools.partial(_gather_body, n_chunks, chunk, width)
    return pl.kernel(
        body,
        out_type=jax.ShapeDtypeStruct((n_rows, width), jnp.float32),
        mesh=mesh,
        scratch_types=[
            pltpu.VMEM((_NBUF, chunk), jnp.int32),
            pltpu.VMEM((_NBUF, chunk), jnp.int32),
            pltpu.VMEM((_NBUF, chunk, pair_width), jnp.float32),
            pltpu.VMEM((_NBUF, chunk, width), jnp.float32),
            pltpu.SemaphoreType.DMA((_NBUF,)),
            pltpu.SemaphoreType.DMA((_NBUF,)),
        ],
        compiler_params=pltpu.CompilerParams(use_tc_tiling_on_sc=True),
    )(table_pairs, idx_flat)


def kernel(inputs, embeddings):
    n_rows = inputs.shape[0] * inputs.shape[1]
    width = embeddings.shape[1]
    idx_flat = jnp.reshape(inputs, (n_rows,)).astype(jnp.int32)
    table_pairs = jnp.reshape(embeddings,
                              (embeddings.shape[0] // 2, 2 * width))
    out = _sc_embedding_lookup(idx_flat, table_pairs, n_rows=n_rows,
                               chunk=128)
    return jnp.reshape(out, inputs.shape + (width,))
